# Initial kernel scaffold; baseline (speedup 1.0000x reference)
import jax
import jax.numpy as jnp
from jax.experimental import pallas as pl


def kernel(x, edge_index, params):
    raise NotImplementedError("write your pallas kernel here")


# speedup vs baseline: 3.9937x; 3.9937x over previous
import functools

import jax
import jax.numpy as jnp
from jax import lax
from jax.experimental import pallas as pl
from jax.experimental.pallas import tpu as pltpu
from jax.experimental.pallas import tpu_sc as plsc

NSC = 2
NSUB = 16
CH = 80


def _quarters(d):
    nq = 2 if d <= 224 else 4
    dq = 16 * ((d + 16 * nq - 1) // (16 * nq))
    return nq, dq


@functools.lru_cache(maxsize=None)
def _make_spmm(n, e, nq, dq, qoff):
    ept = e // NSUB
    nch = ept // CH
    rows_pt = n // NSUB
    ic = rows_pt // 25
    mesh = plsc.VectorSubcoreMesh(
        core_axis_name="c", subcore_axis_name="s",
        num_cores=NSC, num_subcores=NSUB,
    )

    def body(g_hbm, src_hbm, dst_hbm, out_hbm, acc):
        c = lax.axis_index("c")
        s = lax.axis_index("s")
        q = qoff + c
        r0 = s * rows_pt

        def scoped(src_ch, dst_ch, rows_v, init_v, sem):
            def initc(t, carry):
                o = r0 + t * ic
                pltpu.async_copy(g_hbm.at[q].at[pl.ds(o, ic)], init_v,
                                 sem).wait()
                pltpu.sync_copy(init_v, acc.at[pl.ds(o, ic)])
                return carry

            lax.fori_loop(0, 25, initc, 0)
            plsc.subcore_barrier()

            def chunk(j, carry):
                off = s * ept + j * CH
                pltpu.sync_copy(src_hbm.at[pl.ds(off, CH)], src_ch)
                pltpu.sync_copy(dst_hbm.at[pl.ds(off, CH)], dst_ch)
                pltpu.async_copy(g_hbm.at[q].at[src_ch], rows_v, sem).wait()
                pltpu.sync_copy(rows_v, acc.at[dst_ch], add=True)
                return carry

            lax.fori_loop(0, nch, chunk, 0)
            plsc.subcore_barrier()

            def writec(t, carry):
                o = r0 + t * ic
                pltpu.sync_copy(acc.at[pl.ds(o, ic)], init_v)
                pltpu.async_copy(init_v, out_hbm.at[c].at[pl.ds(o, ic)],
                                 sem).wait()
                return carry

            lax.fori_loop(0, 25, writec, 0)

        pl.run_scoped(
            scoped,
            pltpu.VMEM((CH,), jnp.int32),
            pltpu.VMEM((CH,), jnp.int32),
            pltpu.VMEM((CH, dq), jnp.float32),
            pltpu.VMEM((ic, dq), jnp.float32),
            pltpu.SemaphoreType.DMA,
        )

    return pl.kernel(
        body,
        out_type=jax.ShapeDtypeStruct((NSC, n, dq), jnp.float32),
        mesh=mesh,
        compiler_params=pltpu.CompilerParams(use_tc_tiling_on_sc=False),
        scratch_types=[
            pltpu.VMEM_SHARED((n, dq), jnp.float32),
        ],
    )


def _dinv_body(deg_ref, out_ref):
    out_ref[...] = lax.rsqrt(deg_ref[0, :, 0:1])


def _mm_body(a_ref, w_ref, dinv_ref, out_ref):
    a = a_ref[...] * dinv_ref[...]
    out_ref[0] = jnp.dot(a, w_ref[0], preferred_element_type=jnp.float32)


def _mmsplit_body(*refs):
    a_parts, (w_ref, out_ref) = refs[:-2], refs[-2:]
    cols = []
    for ap in a_parts:
        cols.append(ap[0])
        cols.append(ap[1])
    a = jnp.concatenate(cols, axis=1)
    out_ref[0] = jnp.dot(a, w_ref[0], preferred_element_type=jnp.float32)


def _epi_body(s_ref, dinv_ref, b_ref, gam_ref, bet_ref, out_ref):
    dinv = dinv_ref[...]
    z = s_ref[0] * dinv + b_ref[0]
    el = jnp.where(z > 0, z, jnp.exp(jnp.minimum(z, 0.0)) - 1.0)
    mean = jnp.mean(el, axis=0, keepdims=True)
    var = jnp.mean(jnp.square(el - mean), axis=0, keepdims=True)
    bn = (el - mean) * lax.rsqrt(var + 1e-5) * gam_ref[0] + bet_ref[0]
    out_ref[0] = bn * dinv


def _fin_body(s_ref, dinv_ref, b_ref, out_ref):
    out_ref[0] = s_ref[0] * dinv_ref[...] + b_ref[0]


@functools.lru_cache(maxsize=None)
def _make_mm(n, k, nq, dq, bn):
    nb = n // bn
    return pl.pallas_call(
        _mm_body,
        grid=(nq, nb),
        in_specs=[
            pl.BlockSpec((bn, k), lambda q, i: (i, 0)),
            pl.BlockSpec((1, k, dq), lambda q, i: (q, 0, 0)),
            pl.BlockSpec((bn, 1), lambda q, i: (i, 0)),
        ],
        out_specs=pl.BlockSpec((1, bn, dq), lambda q, i: (q, i, 0)),
        out_shape=jax.ShapeDtypeStruct((nq, n, dq), jnp.float32),
    )


@functools.lru_cache(maxsize=None)
def _make_mmsplit(n, nparts, dqp, nq, dq, bn):
    nb = n // bn
    k = nparts * 2 * dqp
    in_specs = [
        pl.BlockSpec((NSC, bn, dqp), lambda q, i: (0, i, 0))
        for _ in range(nparts)
    ]
    in_specs += [
        pl.BlockSpec((1, k, dq), lambda q, i: (q, 0, 0)),
    ]
    return pl.pallas_call(
        _mmsplit_body,
        grid=(nq, nb),
        in_specs=in_specs,
        out_specs=pl.BlockSpec((1, bn, dq), lambda q, i: (q, i, 0)),
        out_shape=jax.ShapeDtypeStruct((nq, n, dq), jnp.float32),
    )


@functools.lru_cache(maxsize=None)
def _make_epi(n, dq):
    return pl.pallas_call(
        _epi_body,
        grid=(NSC,),
        in_specs=[
            pl.BlockSpec((1, n, dq), lambda c: (c, 0, 0)),
            pl.BlockSpec((n, 1), lambda c: (0, 0)),
            pl.BlockSpec((1, 1, dq), lambda c: (c, 0, 0)),
            pl.BlockSpec((1, 1, dq), lambda c: (c, 0, 0)),
            pl.BlockSpec((1, 1, dq), lambda c: (c, 0, 0)),
        ],
        out_specs=pl.BlockSpec((1, n, dq), lambda c: (c, 0, 0)),
        out_shape=jax.ShapeDtypeStruct((NSC, n, dq), jnp.float32),
    )


@functools.lru_cache(maxsize=None)
def _make_fin(n, dq):
    return pl.pallas_call(
        _fin_body,
        grid=(NSC,),
        in_specs=[
            pl.BlockSpec((1, n, dq), lambda c: (c, 0, 0)),
            pl.BlockSpec((n, 1), lambda c: (0, 0)),
            pl.BlockSpec((1, 1, dq), lambda c: (c, 0, 0)),
        ],
        out_specs=pl.BlockSpec((1, n, dq), lambda c: (c, 0, 0)),
        out_shape=jax.ShapeDtypeStruct((NSC, n, dq), jnp.float32),
    )


@functools.lru_cache(maxsize=None)
def _make_dinv(n):
    return pl.pallas_call(
        _dinv_body,
        out_shape=jax.ShapeDtypeStruct((n, 1), jnp.float32),
    )


def _pad_quarters(v, d_out, nq, dq):
    return jnp.pad(v, (0, nq * dq - d_out)).reshape(nq, 1, dq)


def kernel(x, edge_index, params):
    n = x.shape[0]
    e = edge_index.shape[1]
    src, dst = edge_index[0], edge_index[1]
    Ws, bs = params["Ws"], params["bs"]
    gammas, betas = params["gammas"], params["betas"]
    nl = len(Ws)

    ones_g = jnp.ones((NSC, n, 16), jnp.float32)
    deg16 = _make_spmm(n, e, 2, 16, 0)(ones_g, src, dst)
    dinv = _make_dinv(n)(deg16)

    a_parts = None
    dqp = 0
    out = None
    for i in range(nl):
        d_out = Ws[i].shape[1]
        nq, dq = _quarters(d_out)
        k = x.shape[1] if i == 0 else len(a_parts) * 2 * dqp
        w = Ws[i]
        wp = jnp.pad(w, ((0, k - w.shape[0]), (0, nq * dq - d_out)))
        wsplit = wp.reshape(k, nq, dq).transpose(1, 0, 2)
        bp = _pad_quarters(bs[i], d_out, nq, dq)
        if i == 0:
            g = _make_mm(n, k, nq, dq, 2000)(x, wsplit, dinv)
        else:
            g = _make_mmsplit(n, len(a_parts), dqp, nq, dq, 2000)(
                *a_parts, wsplit)
        s_parts = [
            _make_spmm(n, e, nq, dq, p * NSC)(g, src, dst)
            for p in range(nq // NSC)
        ]
        if i < nl - 1:
            gp = _pad_quarters(gammas[i], d_out, nq, dq)
            bep = _pad_quarters(betas[i], d_out, nq, dq)
            epi = _make_epi(n, dq)
            a_parts = [
                epi(s_parts[p], dinv,
                    bp[p * NSC:(p + 1) * NSC],
                    gp[p * NSC:(p + 1) * NSC],
                    bep[p * NSC:(p + 1) * NSC])
                for p in range(nq // NSC)
            ]
            dqp = dq
        else:
            out = _make_fin(n, dq)(s_parts[0], dinv, bp[0:NSC])
    full = jnp.concatenate([out[0], out[1]], axis=1)
    return full[:, :Ws[-1].shape[1]]


# speedup vs baseline: 9.2935x; 2.3270x over previous
import functools

import jax
import jax.numpy as jnp
from jax import lax
from jax.experimental import pallas as pl
from jax.experimental.pallas import tpu as pltpu
from jax.experimental.pallas import tpu_sc as plsc

NSC = 2
NSUB = 16
CH = 80


def _quarters(d):
    nq = 2 if d <= 224 else 4
    dq = 16 * ((d + 16 * nq - 1) // (16 * nq))
    return nq, dq


BLK = 4000


@functools.lru_cache(maxsize=None)
def _make_spmm(n, e, nq, dq, qoff):
    ept = e // NSUB
    nblk = ept // BLK
    nchb = BLK // CH
    rows_pt = n // NSUB
    ic = rows_pt // 25
    mesh = plsc.VectorSubcoreMesh(
        core_axis_name="c", subcore_axis_name="s",
        num_cores=NSC, num_subcores=NSUB,
    )

    def body(g_hbm, src_hbm, dst_hbm, out_hbm, acc):
        c = lax.axis_index("c")
        s = lax.axis_index("s")
        q = qoff + c
        r0 = s * rows_pt

        def scoped(src_blk, dst_blk, dst_ch0, dst_ch1, rows0, rows1, init_v,
                   sem0, sem1, semi):
            def gather(t, rows, sem):
                pltpu.async_copy(
                    g_hbm.at[q].at[src_blk.at[pl.ds(t * CH, CH)]], rows, sem)

            def gwait(rows, sem):
                pltpu.make_async_copy(
                    g_hbm.at[q].at[src_blk.at[pl.ds(0, CH)]], rows, sem).wait()

            def scatter(t, rows, dst_ch):
                for u in range(CH // 16):
                    dst_ch[pl.ds(u * 16, 16)] = (
                        dst_blk[pl.ds(t * CH + u * 16, 16)])
                pltpu.sync_copy(rows, acc.at[dst_ch], add=True)

            def initc(t, carry):
                o = r0 + t * ic
                pltpu.async_copy(g_hbm.at[q].at[pl.ds(o, ic)], init_v,
                                 semi).wait()
                pltpu.sync_copy(init_v, acc.at[pl.ds(o, ic)])
                return carry

            lax.fori_loop(0, 25, initc, 0)
            plsc.subcore_barrier()

            def blockf(b, carry):
                boff = s * ept + b * BLK
                pltpu.sync_copy(src_hbm.at[pl.ds(boff, BLK)], src_blk)
                pltpu.sync_copy(dst_hbm.at[pl.ds(boff, BLK)], dst_blk)
                gather(0, rows0, sem0)

                def inner(t, carry2):
                    c0 = 2 * t
                    gather(c0 + 1, rows1, sem1)
                    gwait(rows0, sem0)
                    scatter(c0, rows0, dst_ch0)

                    @pl.when(t < nchb // 2 - 1)
                    def _():
                        gather(c0 + 2, rows0, sem0)

                    gwait(rows1, sem1)
                    scatter(c0 + 1, rows1, dst_ch1)
                    return carry2

                lax.fori_loop(0, nchb // 2, inner, 0)
                return carry

            lax.fori_loop(0, nblk, blockf, 0)
            plsc.subcore_barrier()

            def writec(t, carry):
                o = r0 + t * ic
                pltpu.sync_copy(acc.at[pl.ds(o, ic)], init_v)
                pltpu.async_copy(init_v, out_hbm.at[c].at[pl.ds(o, ic)],
                                 semi).wait()
                return carry

            lax.fori_loop(0, 25, writec, 0)

        pl.run_scoped(
            scoped,
            pltpu.VMEM((BLK,), jnp.int32),
            pltpu.VMEM((BLK,), jnp.int32),
            pltpu.VMEM((CH,), jnp.int32),
            pltpu.VMEM((CH,), jnp.int32),
            pltpu.VMEM((CH, dq), jnp.float32),
            pltpu.VMEM((CH, dq), jnp.float32),
            pltpu.VMEM((ic, dq), jnp.float32),
            pltpu.SemaphoreType.DMA,
            pltpu.SemaphoreType.DMA,
            pltpu.SemaphoreType.DMA,
        )

    return pl.kernel(
        body,
        out_type=jax.ShapeDtypeStruct((NSC, n, dq), jnp.float32),
        mesh=mesh,
        compiler_params=pltpu.CompilerParams(use_tc_tiling_on_sc=False),
        scratch_types=[
            pltpu.VMEM_SHARED((n, dq), jnp.float32),
        ],
    )


def _dinv_body(deg_ref, out_ref):
    out_ref[...] = lax.rsqrt(deg_ref[0, :, 0:1])


def _mm_body(a_ref, w_ref, dinv_ref, out_ref):
    a = a_ref[...] * dinv_ref[...]
    out_ref[0] = jnp.dot(a, w_ref[0], preferred_element_type=jnp.float32)


def _mmsplit_body(*refs):
    a_parts, (w_ref, out_ref) = refs[:-2], refs[-2:]
    cols = []
    for ap in a_parts:
        cols.append(ap[0])
        cols.append(ap[1])
    a = jnp.concatenate(cols, axis=1)
    out_ref[0] = jnp.dot(a, w_ref[0], preferred_element_type=jnp.float32)


def _epi_body(s_ref, dinv_ref, b_ref, gam_ref, bet_ref, out_ref):
    dinv = dinv_ref[...]
    z = s_ref[0] * dinv + b_ref[0]
    el = jnp.where(z > 0, z, jnp.exp(jnp.minimum(z, 0.0)) - 1.0)
    mean = jnp.mean(el, axis=0, keepdims=True)
    var = jnp.mean(jnp.square(el - mean), axis=0, keepdims=True)
    bn = (el - mean) * lax.rsqrt(var + 1e-5) * gam_ref[0] + bet_ref[0]
    out_ref[0] = bn * dinv


def _fin_body(s_ref, dinv_ref, b_ref, out_ref):
    out_ref[0] = s_ref[0] * dinv_ref[...] + b_ref[0]


@functools.lru_cache(maxsize=None)
def _make_mm(n, k, nq, dq, bn):
    nb = n // bn
    return pl.pallas_call(
        _mm_body,
        grid=(nq, nb),
        in_specs=[
            pl.BlockSpec((bn, k), lambda q, i: (i, 0)),
            pl.BlockSpec((1, k, dq), lambda q, i: (q, 0, 0)),
            pl.BlockSpec((bn, 1), lambda q, i: (i, 0)),
        ],
        out_specs=pl.BlockSpec((1, bn, dq), lambda q, i: (q, i, 0)),
        out_shape=jax.ShapeDtypeStruct((nq, n, dq), jnp.float32),
    )


@functools.lru_cache(maxsize=None)
def _make_mmsplit(n, nparts, dqp, nq, dq, bn):
    nb = n // bn
    k = nparts * 2 * dqp
    in_specs = [
        pl.BlockSpec((NSC, bn, dqp), lambda q, i: (0, i, 0))
        for _ in range(nparts)
    ]
    in_specs += [
        pl.BlockSpec((1, k, dq), lambda q, i: (q, 0, 0)),
    ]
    return pl.pallas_call(
        _mmsplit_body,
        grid=(nq, nb),
        in_specs=in_specs,
        out_specs=pl.BlockSpec((1, bn, dq), lambda q, i: (q, i, 0)),
        out_shape=jax.ShapeDtypeStruct((nq, n, dq), jnp.float32),
    )


@functools.lru_cache(maxsize=None)
def _make_epi(n, dq):
    return pl.pallas_call(
        _epi_body,
        grid=(NSC,),
        in_specs=[
            pl.BlockSpec((1, n, dq), lambda c: (c, 0, 0)),
            pl.BlockSpec((n, 1), lambda c: (0, 0)),
            pl.BlockSpec((1, 1, dq), lambda c: (c, 0, 0)),
            pl.BlockSpec((1, 1, dq), lambda c: (c, 0, 0)),
            pl.BlockSpec((1, 1, dq), lambda c: (c, 0, 0)),
        ],
        out_specs=pl.BlockSpec((1, n, dq), lambda c: (c, 0, 0)),
        out_shape=jax.ShapeDtypeStruct((NSC, n, dq), jnp.float32),
    )


@functools.lru_cache(maxsize=None)
def _make_fin(n, dq):
    return pl.pallas_call(
        _fin_body,
        grid=(NSC,),
        in_specs=[
            pl.BlockSpec((1, n, dq), lambda c: (c, 0, 0)),
            pl.BlockSpec((n, 1), lambda c: (0, 0)),
            pl.BlockSpec((1, 1, dq), lambda c: (c, 0, 0)),
        ],
        out_specs=pl.BlockSpec((1, n, dq), lambda c: (c, 0, 0)),
        out_shape=jax.ShapeDtypeStruct((NSC, n, dq), jnp.float32),
    )


@functools.lru_cache(maxsize=None)
def _make_dinv(n):
    return pl.pallas_call(
        _dinv_body,
        out_shape=jax.ShapeDtypeStruct((n, 1), jnp.float32),
    )


def _pad_quarters(v, d_out, nq, dq):
    return jnp.pad(v, (0, nq * dq - d_out)).reshape(nq, 1, dq)


def kernel(x, edge_index, params):
    n = x.shape[0]
    e = edge_index.shape[1]
    src, dst = edge_index[0], edge_index[1]
    Ws, bs = params["Ws"], params["bs"]
    gammas, betas = params["gammas"], params["betas"]
    nl = len(Ws)

    ones_g = jnp.ones((NSC, n, 16), jnp.float32)
    deg16 = _make_spmm(n, e, 2, 16, 0)(ones_g, src, dst)
    dinv = _make_dinv(n)(deg16)

    a_parts = None
    dqp = 0
    out = None
    for i in range(nl):
        d_out = Ws[i].shape[1]
        nq, dq = _quarters(d_out)
        k = x.shape[1] if i == 0 else len(a_parts) * 2 * dqp
        w = Ws[i]
        wp = jnp.pad(w, ((0, k - w.shape[0]), (0, nq * dq - d_out)))
        wsplit = wp.reshape(k, nq, dq).transpose(1, 0, 2)
        bp = _pad_quarters(bs[i], d_out, nq, dq)
        if i == 0:
            g = _make_mm(n, k, nq, dq, 2000)(x, wsplit, dinv)
        else:
            g = _make_mmsplit(n, len(a_parts), dqp, nq, dq, 2000)(
                *a_parts, wsplit)
        s_parts = [
            _make_spmm(n, e, nq, dq, p * NSC)(g, src, dst)
            for p in range(nq // NSC)
        ]
        if i < nl - 1:
            gp = _pad_quarters(gammas[i], d_out, nq, dq)
            bep = _pad_quarters(betas[i], d_out, nq, dq)
            epi = _make_epi(n, dq)
            a_parts = [
                epi(s_parts[p], dinv,
                    bp[p * NSC:(p + 1) * NSC],
                    gp[p * NSC:(p + 1) * NSC],
                    bep[p * NSC:(p + 1) * NSC])
                for p in range(nq // NSC)
            ]
            dqp = dq
        else:
            out = _make_fin(n, dq)(s_parts[0], dinv, bp[0:NSC])
    full = jnp.concatenate([out[0], out[1]], axis=1)
    return full[:, :Ws[-1].shape[1]]


# speedup vs baseline: 10.2087x; 1.0985x over previous
import functools

import jax
import jax.numpy as jnp
from jax import lax
from jax.experimental import pallas as pl
from jax.experimental.pallas import tpu as pltpu
from jax.experimental.pallas import tpu_sc as plsc

NSC = 2
NSUB = 16
CH = 80


def _quarters(d):
    nq = 2 if d <= 224 else 4
    dq = 16 * ((d + 16 * nq - 1) // (16 * nq))
    return nq, dq


BLK = 20000


@functools.lru_cache(maxsize=None)
def _make_spmm(n, e, nq, dq, qoff):
    ept = e // NSUB
    nblk = ept // BLK
    nchb = BLK // CH
    rows_pt = n // NSUB
    ic = rows_pt // 25
    mesh = plsc.VectorSubcoreMesh(
        core_axis_name="c", subcore_axis_name="s",
        num_cores=NSC, num_subcores=NSUB,
    )

    def body(g_hbm, src_hbm, dst_hbm, out_hbm, acc):
        c = lax.axis_index("c")
        s = lax.axis_index("s")
        q = qoff + c
        r0 = s * rows_pt

        def scoped(src_blk, dst_blk, dst_ch0, dst_ch1, rows0, rows1, init_v,
                   sem0, sem1, semi):
            def gather(t, rows, sem):
                pltpu.async_copy(
                    g_hbm.at[q].at[src_blk.at[pl.ds(t * CH, CH)]], rows, sem)

            def gwait(rows, sem):
                pltpu.make_async_copy(
                    g_hbm.at[q].at[src_blk.at[pl.ds(0, CH)]], rows, sem).wait()

            def scatter(t, rows, dst_ch):
                for u in range(CH // 16):
                    dst_ch[pl.ds(u * 16, 16)] = (
                        dst_blk[pl.ds(t * CH + u * 16, 16)])
                pltpu.sync_copy(rows, acc.at[dst_ch], add=True)

            z16 = jnp.zeros((16,), jnp.float32)
            for u in range(dq // 16):
                init_v[0, pl.ds(u * 16, 16)] = z16
            for r in range(1, ic):
                for u in range(dq // 16):
                    init_v[r, pl.ds(u * 16, 16)] = z16

            def initc(t, carry):
                pltpu.sync_copy(init_v, acc.at[pl.ds(r0 + t * ic, ic)])
                return carry

            lax.fori_loop(0, 25, initc, 0)
            plsc.subcore_barrier()

            def blockf(b, carry):
                boff = s * ept + b * BLK
                pltpu.sync_copy(src_hbm.at[pl.ds(boff, BLK)], src_blk)
                pltpu.sync_copy(dst_hbm.at[pl.ds(boff, BLK)], dst_blk)
                gather(0, rows0, sem0)

                def inner(t, carry2):
                    c0 = 2 * t
                    gather(c0 + 1, rows1, sem1)
                    gwait(rows0, sem0)
                    scatter(c0, rows0, dst_ch0)

                    @pl.when(t < nchb // 2 - 1)
                    def _():
                        gather(c0 + 2, rows0, sem0)

                    gwait(rows1, sem1)
                    scatter(c0 + 1, rows1, dst_ch1)
                    return carry2

                lax.fori_loop(0, nchb // 2, inner, 0)
                return carry

            lax.fori_loop(0, nblk, blockf, 0)
            plsc.subcore_barrier()

            def writec(t, carry):
                o = r0 + t * ic
                pltpu.sync_copy(acc.at[pl.ds(o, ic)], init_v)
                pltpu.async_copy(init_v, out_hbm.at[c].at[pl.ds(o, ic)],
                                 semi).wait()
                return carry

            lax.fori_loop(0, 25, writec, 0)

        pl.run_scoped(
            scoped,
            pltpu.VMEM((BLK,), jnp.int32),
            pltpu.VMEM((BLK,), jnp.int32),
            pltpu.VMEM((CH,), jnp.int32),
            pltpu.VMEM((CH,), jnp.int32),
            pltpu.VMEM((CH, dq), jnp.float32),
            pltpu.VMEM((CH, dq), jnp.float32),
            pltpu.VMEM((ic, dq), jnp.float32),
            pltpu.SemaphoreType.DMA,
            pltpu.SemaphoreType.DMA,
            pltpu.SemaphoreType.DMA,
        )

    return pl.kernel(
        body,
        out_type=jax.ShapeDtypeStruct((NSC, n, dq), jnp.float32),
        mesh=mesh,
        compiler_params=pltpu.CompilerParams(use_tc_tiling_on_sc=False),
        scratch_types=[
            pltpu.VMEM_SHARED((n, dq), jnp.float32),
        ],
    )


def _dinv_body(deg_ref, out_ref):
    out_ref[...] = lax.rsqrt(deg_ref[0, :, 0:1] + 1.0)


def _mm_body(a_ref, w_ref, dinv_ref, out_ref):
    a = a_ref[...] * dinv_ref[...]
    out_ref[0] = jnp.dot(a, w_ref[0], preferred_element_type=jnp.float32)


def _mmsplit_body(*refs):
    a_parts, (w_ref, out_ref) = refs[:-2], refs[-2:]
    cols = []
    for ap in a_parts:
        cols.append(ap[0])
        cols.append(ap[1])
    a = jnp.concatenate(cols, axis=1)
    out_ref[0] = jnp.dot(a, w_ref[0], preferred_element_type=jnp.float32)


def _epi_body(s_ref, g_ref, dinv_ref, b_ref, gam_ref, bet_ref, out_ref):
    dinv = dinv_ref[...]
    z = (s_ref[0] + g_ref[0]) * dinv + b_ref[0]
    el = jnp.where(z > 0, z, jnp.exp(jnp.minimum(z, 0.0)) - 1.0)
    mean = jnp.mean(el, axis=0, keepdims=True)
    var = jnp.mean(jnp.square(el - mean), axis=0, keepdims=True)
    bn = (el - mean) * lax.rsqrt(var + 1e-5) * gam_ref[0] + bet_ref[0]
    out_ref[0] = bn * dinv


def _fin_body(s_ref, g_ref, dinv_ref, b_ref, out_ref):
    out_ref[0] = (s_ref[0] + g_ref[0]) * dinv_ref[...] + b_ref[0]


@functools.lru_cache(maxsize=None)
def _make_mm(n, k, nq, dq, bn):
    nb = n // bn
    return pl.pallas_call(
        _mm_body,
        grid=(nq, nb),
        in_specs=[
            pl.BlockSpec((bn, k), lambda q, i: (i, 0)),
            pl.BlockSpec((1, k, dq), lambda q, i: (q, 0, 0)),
            pl.BlockSpec((bn, 1), lambda q, i: (i, 0)),
        ],
        out_specs=pl.BlockSpec((1, bn, dq), lambda q, i: (q, i, 0)),
        out_shape=jax.ShapeDtypeStruct((nq, n, dq), jnp.float32),
    )


@functools.lru_cache(maxsize=None)
def _make_mmsplit(n, nparts, dqp, nq, dq, bn):
    nb = n // bn
    k = nparts * 2 * dqp
    in_specs = [
        pl.BlockSpec((NSC, bn, dqp), lambda q, i: (0, i, 0))
        for _ in range(nparts)
    ]
    in_specs += [
        pl.BlockSpec((1, k, dq), lambda q, i: (q, 0, 0)),
    ]
    return pl.pallas_call(
        _mmsplit_body,
        grid=(nq, nb),
        in_specs=in_specs,
        out_specs=pl.BlockSpec((1, bn, dq), lambda q, i: (q, i, 0)),
        out_shape=jax.ShapeDtypeStruct((nq, n, dq), jnp.float32),
    )


@functools.lru_cache(maxsize=None)
def _make_epi(n, dq, qoff):
    return pl.pallas_call(
        _epi_body,
        grid=(NSC,),
        in_specs=[
            pl.BlockSpec((1, n, dq), lambda c: (c, 0, 0)),
            pl.BlockSpec((1, n, dq), lambda c: (qoff + c, 0, 0)),
            pl.BlockSpec((n, 1), lambda c: (0, 0)),
            pl.BlockSpec((1, 1, dq), lambda c: (c, 0, 0)),
            pl.BlockSpec((1, 1, dq), lambda c: (c, 0, 0)),
            pl.BlockSpec((1, 1, dq), lambda c: (c, 0, 0)),
        ],
        out_specs=pl.BlockSpec((1, n, dq), lambda c: (c, 0, 0)),
        out_shape=jax.ShapeDtypeStruct((NSC, n, dq), jnp.float32),
    )


@functools.lru_cache(maxsize=None)
def _make_fin(n, dq):
    return pl.pallas_call(
        _fin_body,
        grid=(NSC,),
        in_specs=[
            pl.BlockSpec((1, n, dq), lambda c: (c, 0, 0)),
            pl.BlockSpec((1, n, dq), lambda c: (c, 0, 0)),
            pl.BlockSpec((n, 1), lambda c: (0, 0)),
            pl.BlockSpec((1, 1, dq), lambda c: (c, 0, 0)),
        ],
        out_specs=pl.BlockSpec((1, n, dq), lambda c: (c, 0, 0)),
        out_shape=jax.ShapeDtypeStruct((NSC, n, dq), jnp.float32),
    )


@functools.lru_cache(maxsize=None)
def _make_dinv(n):
    return pl.pallas_call(
        _dinv_body,
        out_shape=jax.ShapeDtypeStruct((n, 1), jnp.float32),
    )


def _pad_quarters(v, d_out, nq, dq):
    return jnp.pad(v, (0, nq * dq - d_out)).reshape(nq, 1, dq)


def kernel(x, edge_index, params):
    n = x.shape[0]
    e = edge_index.shape[1]
    src, dst = edge_index[0], edge_index[1]
    Ws, bs = params["Ws"], params["bs"]
    gammas, betas = params["gammas"], params["betas"]
    nl = len(Ws)

    ones_g = jnp.ones((NSC, n, 16), jnp.float32)
    deg16 = _make_spmm(n, e, 2, 16, 0)(ones_g, src, dst)
    dinv = _make_dinv(n)(deg16)

    a_parts = None
    dqp = 0
    out = None
    for i in range(nl):
        d_out = Ws[i].shape[1]
        nq, dq = _quarters(d_out)
        k = x.shape[1] if i == 0 else len(a_parts) * 2 * dqp
        w = Ws[i]
        wp = jnp.pad(w, ((0, k - w.shape[0]), (0, nq * dq - d_out)))
        wsplit = wp.reshape(k, nq, dq).transpose(1, 0, 2)
        bp = _pad_quarters(bs[i], d_out, nq, dq)
        if i == 0:
            g = _make_mm(n, k, nq, dq, 2000)(x, wsplit, dinv)
        else:
            g = _make_mmsplit(n, len(a_parts), dqp, nq, dq, 2000)(
                *a_parts, wsplit)
        s_parts = [
            _make_spmm(n, e, nq, dq, p * NSC)(g, src, dst)
            for p in range(nq // NSC)
        ]
        if i < nl - 1:
            gp = _pad_quarters(gammas[i], d_out, nq, dq)
            bep = _pad_quarters(betas[i], d_out, nq, dq)
            a_parts = [
                _make_epi(n, dq, p * NSC)(
                    s_parts[p], g, dinv,
                    bp[p * NSC:(p + 1) * NSC],
                    gp[p * NSC:(p + 1) * NSC],
                    bep[p * NSC:(p + 1) * NSC])
                for p in range(nq // NSC)
            ]
            dqp = dq
        else:
            out = _make_fin(n, dq)(s_parts[0], g, dinv, bp[0:NSC])
    full = jnp.concatenate([out[0], out[1]], axis=1)
    return full[:, :Ws[-1].shape[1]]


# speedup vs baseline: 11.2116x; 1.0982x over previous
import functools

import jax
import jax.numpy as jnp
from jax import lax
from jax.experimental import pallas as pl
from jax.experimental.pallas import tpu as pltpu
from jax.experimental.pallas import tpu_sc as plsc

NSC = 2
NSUB = 16
CH = 80


def _pairs(d):
    if d <= 224:
        return (16 * ((d + 31) // 32),)
    assert d <= 448
    return (112, 16 * ((d - 224 + 31) // 32))


_SC_MESH = None


def _mesh():
    global _SC_MESH
    if _SC_MESH is None:
        _SC_MESH = plsc.VectorSubcoreMesh(
            core_axis_name="c", subcore_axis_name="s",
            num_cores=NSC, num_subcores=NSUB,
        )
    return _SC_MESH


@functools.lru_cache(maxsize=None)
def _make_spmm(n, e, w):
    ept = e // NSUB
    nchb = ept // CH
    rows_pt = n // NSUB
    ic = rows_pt // 25

    def body(g_hbm, src_hbm, dst_hbm, out_hbm, acc):
        c = lax.axis_index("c")
        s = lax.axis_index("s")
        r0 = s * rows_pt

        def scoped(src_blk, dst_blk, dst_ch0, dst_ch1, rows0, rows1, init_v,
                   sem0, sem1, semi):
            def gather(t, rows, sem):
                pltpu.async_copy(
                    g_hbm.at[c].at[src_blk.at[pl.ds(t * CH, CH)]], rows, sem)

            def gwait(rows, sem):
                pltpu.make_async_copy(
                    g_hbm.at[c].at[src_blk.at[pl.ds(0, CH)]], rows, sem).wait()

            def scatter(t, rows, dst_ch):
                for u in range(CH // 16):
                    dst_ch[pl.ds(u * 16, 16)] = (
                        dst_blk[pl.ds(t * CH + u * 16, 16)])
                pltpu.sync_copy(rows, acc.at[dst_ch], add=True)

            z16 = jnp.zeros((16,), jnp.float32)
            for r in range(ic):
                for u in range(w // 16):
                    init_v[r, pl.ds(u * 16, 16)] = z16

            def initc(t, carry):
                pltpu.sync_copy(init_v, acc.at[pl.ds(r0 + t * ic, ic)])
                return carry

            lax.fori_loop(0, 25, initc, 0)
            pltpu.sync_copy(src_hbm.at[pl.ds(s * ept, ept)], src_blk)
            pltpu.sync_copy(dst_hbm.at[pl.ds(s * ept, ept)], dst_blk)
            plsc.subcore_barrier()

            gather(0, rows0, sem0)

            def inner(t, carry2):
                c0 = 2 * t
                gather(c0 + 1, rows1, sem1)
                gwait(rows0, sem0)
                scatter(c0, rows0, dst_ch0)

                @pl.when(t < nchb // 2 - 1)
                def _():
                    gather(c0 + 2, rows0, sem0)

                gwait(rows1, sem1)
                scatter(c0 + 1, rows1, dst_ch1)
                return carry2

            lax.fori_loop(0, nchb // 2, inner, 0)
            plsc.subcore_barrier()

            def writec(t, carry):
                o = r0 + t * ic
                pltpu.sync_copy(acc.at[pl.ds(o, ic)], init_v)
                pltpu.async_copy(init_v, out_hbm.at[c].at[pl.ds(o, ic)],
                                 semi).wait()
                return carry

            lax.fori_loop(0, 25, writec, 0)

        pl.run_scoped(
            scoped,
            pltpu.VMEM((ept,), jnp.int32),
            pltpu.VMEM((ept,), jnp.int32),
            pltpu.VMEM((CH,), jnp.int32),
            pltpu.VMEM((CH,), jnp.int32),
            pltpu.VMEM((CH, w), jnp.float32),
            pltpu.VMEM((CH, w), jnp.float32),
            pltpu.VMEM((ic, w), jnp.float32),
            pltpu.SemaphoreType.DMA,
            pltpu.SemaphoreType.DMA,
            pltpu.SemaphoreType.DMA,
        )

    return pl.kernel(
        body,
        out_type=jax.ShapeDtypeStruct((NSC, n, w), jnp.float32),
        mesh=_mesh(),
        compiler_params=pltpu.CompilerParams(use_tc_tiling_on_sc=False),
        scratch_types=[
            pltpu.VMEM_SHARED((n, w), jnp.float32),
        ],
    )


@functools.lru_cache(maxsize=None)
def _make_deg(n, e):
    ept = e // (NSC * NSUB)
    nch = ept // CH
    rows_pt = n // NSUB
    DW = 16

    def body(dst_hbm, out_hbm, acc):
        c = lax.axis_index("c")
        s = lax.axis_index("s")
        wid = s * NSC + c
        r0 = s * rows_pt

        def scoped(dst_blk, dst_ch, ones_v, init_v, semi):
            z16 = jnp.zeros((16,), jnp.float32)
            o16 = jnp.ones((16,), jnp.float32)
            for r in range(CH):
                ones_v[r, pl.ds(0, 16)] = o16
            for r in range(rows_pt // 25):
                init_v[r, pl.ds(0, 16)] = z16

            def initc(t, carry):
                pltpu.sync_copy(
                    init_v, acc.at[pl.ds(r0 + t * (rows_pt // 25),
                                         rows_pt // 25)])
                return carry

            lax.fori_loop(0, 25, initc, 0)
            pltpu.sync_copy(dst_hbm.at[pl.ds(wid * ept, ept)], dst_blk)
            plsc.subcore_barrier()

            def chunk(j, carry):
                for u in range(CH // 16):
                    dst_ch[pl.ds(u * 16, 16)] = (
                        dst_blk[pl.ds(j * CH + u * 16, 16)])
                pltpu.sync_copy(ones_v, acc.at[dst_ch], add=True)
                return carry

            lax.fori_loop(0, nch, chunk, 0)
            plsc.subcore_barrier()

            def writec(t, carry):
                o = r0 + t * (rows_pt // 25)
                pltpu.sync_copy(acc.at[pl.ds(o, rows_pt // 25)], init_v)
                pltpu.async_copy(
                    init_v, out_hbm.at[c].at[pl.ds(o, rows_pt // 25)],
                    semi).wait()
                return carry

            lax.fori_loop(0, 25, writec, 0)

        pl.run_scoped(
            scoped,
            pltpu.VMEM((ept,), jnp.int32),
            pltpu.VMEM((CH,), jnp.int32),
            pltpu.VMEM((CH, DW), jnp.float32),
            pltpu.VMEM((rows_pt // 25, DW), jnp.float32),
            pltpu.SemaphoreType.DMA,
        )

    return pl.kernel(
        body,
        out_type=jax.ShapeDtypeStruct((NSC, n, DW), jnp.float32),
        mesh=_mesh(),
        compiler_params=pltpu.CompilerParams(use_tc_tiling_on_sc=False),
        scratch_types=[
            pltpu.VMEM_SHARED((n, DW), jnp.float32),
        ],
    )


def _dinv_body(deg_ref, out_ref):
    out_ref[...] = lax.rsqrt(
        deg_ref[0, :, 0:1] + deg_ref[1, :, 0:1] + 1.0)


def _make_mm_body(widths, scale):
    noff = []
    off = 0
    for w in widths:
        noff.append(off)
        off += 2 * w

    def mm_body(*refs):
        if scale:
            a_ref, w_ref, dinv_ref = refs[:3]
            outs = refs[3:]
            a = a_ref[...] * dinv_ref[...]
        else:
            na = len(refs) - 1 - len(widths)
            a_parts, (w_ref,) = refs[:na], refs[na:na + 1]
            outs = refs[na + 1:]
            cols = []
            for ap in a_parts:
                cols.append(ap[0])
                cols.append(ap[1])
            a = jnp.concatenate(cols, axis=1)
        g = jnp.dot(a, w_ref[...], preferred_element_type=jnp.float32)
        for p, w in enumerate(widths):
            outs[p][0] = g[:, noff[p]:noff[p] + w]
            outs[p][1] = g[:, noff[p] + w:noff[p] + 2 * w]

    return mm_body


@functools.lru_cache(maxsize=None)
def _make_mm(n, k, widths, bn):
    nb = n // bn
    wtot = 2 * sum(widths)
    return pl.pallas_call(
        _make_mm_body(widths, True),
        grid=(nb,),
        in_specs=[
            pl.BlockSpec((bn, k), lambda i: (i, 0)),
            pl.BlockSpec((k, wtot), lambda i: (0, 0)),
            pl.BlockSpec((bn, 1), lambda i: (i, 0)),
        ],
        out_specs=[
            pl.BlockSpec((NSC, bn, w), lambda i: (0, i, 0)) for w in widths
        ],
        out_shape=[
            jax.ShapeDtypeStruct((NSC, n, w), jnp.float32) for w in widths
        ],
    )


@functools.lru_cache(maxsize=None)
def _make_mmsplit(n, in_widths, widths, bn):
    nb = n // bn
    k = 2 * sum(in_widths)
    wtot = 2 * sum(widths)
    in_specs = [
        pl.BlockSpec((NSC, bn, w), lambda i: (0, i, 0)) for w in in_widths
    ]
    in_specs.append(pl.BlockSpec((k, wtot), lambda i: (0, 0)))
    return pl.pallas_call(
        _make_mm_body(widths, False),
        grid=(nb,),
        in_specs=in_specs,
        out_specs=[
            pl.BlockSpec((NSC, bn, w), lambda i: (0, i, 0)) for w in widths
        ],
        out_shape=[
            jax.ShapeDtypeStruct((NSC, n, w), jnp.float32) for w in widths
        ],
    )


def _epi_body(s_ref, g_ref, dinv_ref, b_ref, gam_ref, bet_ref, out_ref):
    dinv = dinv_ref[...]
    z = (s_ref[0] + g_ref[0]) * dinv + b_ref[0]
    el = jnp.where(z > 0, z, jnp.exp(jnp.minimum(z, 0.0)) - 1.0)
    mean = jnp.mean(el, axis=0, keepdims=True)
    var = jnp.mean(jnp.square(el - mean), axis=0, keepdims=True)
    bn = (el - mean) * lax.rsqrt(var + 1e-5) * gam_ref[0] + bet_ref[0]
    out_ref[0] = bn * dinv


def _fin_body(s_ref, g_ref, dinv_ref, b_ref, out_ref):
    out_ref[0] = (s_ref[0] + g_ref[0]) * dinv_ref[...] + b_ref[0]


@functools.lru_cache(maxsize=None)
def _make_epi(n, w):
    return pl.pallas_call(
        _epi_body,
        grid=(NSC,),
        in_specs=[
            pl.BlockSpec((1, n, w), lambda c: (c, 0, 0)),
            pl.BlockSpec((1, n, w), lambda c: (c, 0, 0)),
            pl.BlockSpec((n, 1), lambda c: (0, 0)),
            pl.BlockSpec((1, 1, w), lambda c: (c, 0, 0)),
            pl.BlockSpec((1, 1, w), lambda c: (c, 0, 0)),
            pl.BlockSpec((1, 1, w), lambda c: (c, 0, 0)),
        ],
        out_specs=pl.BlockSpec((1, n, w), lambda c: (c, 0, 0)),
        out_shape=jax.ShapeDtypeStruct((NSC, n, w), jnp.float32),
    )


@functools.lru_cache(maxsize=None)
def _make_fin(n, w):
    return pl.pallas_call(
        _fin_body,
        grid=(NSC,),
        in_specs=[
            pl.BlockSpec((1, n, w), lambda c: (c, 0, 0)),
            pl.BlockSpec((1, n, w), lambda c: (c, 0, 0)),
            pl.BlockSpec((n, 1), lambda c: (0, 0)),
            pl.BlockSpec((1, 1, w), lambda c: (c, 0, 0)),
        ],
        out_specs=pl.BlockSpec((1, n, w), lambda c: (c, 0, 0)),
        out_shape=jax.ShapeDtypeStruct((NSC, n, w), jnp.float32),
    )


@functools.lru_cache(maxsize=None)
def _make_dinv(n):
    return pl.pallas_call(
        _dinv_body,
        out_shape=jax.ShapeDtypeStruct((n, 1), jnp.float32),
    )


def _pad_pair(v, d_out, widths, p):
    wtot = 2 * sum(widths)
    off = 2 * sum(widths[:p])
    vp = jnp.pad(v, (0, wtot - d_out))[off:off + 2 * widths[p]]
    return vp.reshape(NSC, 1, widths[p])


def kernel(x, edge_index, params):
    n = x.shape[0]
    e = edge_index.shape[1]
    src, dst = edge_index[0], edge_index[1]
    Ws, bs = params["Ws"], params["bs"]
    gammas, betas = params["gammas"], params["betas"]
    nl = len(Ws)

    deg = _make_deg(n, e)(dst)
    dinv = _make_dinv(n)(deg)

    a_parts = None
    in_widths = None
    out = None
    for i in range(nl):
        d_out = Ws[i].shape[1]
        widths = _pairs(d_out)
        k = x.shape[1] if i == 0 else 2 * sum(in_widths)
        w = Ws[i]
        wp = jnp.pad(w, ((0, k - w.shape[0]), (0, 2 * sum(widths) - d_out)))
        if i == 0:
            g_parts = _make_mm(n, k, widths, 2000)(x, wp, dinv)
        else:
            g_parts = _make_mmsplit(n, in_widths, widths, 2000)(*a_parts, wp)
        s_parts = [
            _make_spmm(n, e, widths[p])(g_parts[p], src, dst)
            for p in range(len(widths))
        ]
        if i < nl - 1:
            a_parts = [
                _make_epi(n, widths[p])(
                    s_parts[p], g_parts[p], dinv,
                    _pad_pair(bs[i], d_out, widths, p),
                    _pad_pair(gammas[i], d_out, widths, p),
                    _pad_pair(betas[i], d_out, widths, p))
                for p in range(len(widths))
            ]
            in_widths = widths
        else:
            out = _make_fin(n, widths[0])(
                s_parts[0], g_parts[0], dinv,
                _pad_pair(bs[i], d_out, widths, 0))
    full = jnp.concatenate([out[0], out[1]], axis=1)
    return full[:, :Ws[-1].shape[1]]
